# baseline (speedup 1.0000x reference)
import functools

import jax
import jax.numpy as jnp
from jax.experimental import pallas as pl
from jax.experimental.pallas import tpu as pltpu

_B = 512
_NPG = 30
_EPG = 60
_G = 2 * _B
_N = _G * _NPG
_E = _G * _EPG
_PROP_STEPS = 3
_SINK_ITERS = 10
_TEMP = 0.1

_P = 8
_NBLK = _B // _P
_NPP = 64
_EPP = 128
_NODE_R = _P * _NPP
_EDGE_R = _P * _EPP
_NEG = -1e30


def _mm(a, b):
    return jax.lax.dot_general(a, b, (((1,), (0,)), ((), ())),
                               preferred_element_type=jnp.float32)


def _mm_bt(a, b):
    return jax.lax.dot_general(a, b, (((1,), (1,)), ((), ())),
                               preferred_element_type=jnp.float32)


def _mm_at(a, b):
    return jax.lax.dot_general(a, b, (((0,), (0,)), ((), ())),
                               preferred_element_type=jnp.float32)


def _lse(x, axis):
    m = jnp.max(x, axis=axis, keepdims=True)
    return m + jnp.log(jnp.sum(jnp.exp(x - m), axis=axis, keepdims=True))


def _block_kernel(nf, ef, fl, tl,
                  Wen, ben, Wee, bee, Wm, bm, Wn, bn, Wp, bp,
                  Wi1, bi1, Wi2, bi2, Ws1, bs1, Ws2, bs2, out):
    f_row = fl[0]
    t_row = tl[0]
    niota = jax.lax.broadcasted_iota(jnp.int32, (_NODE_R, _EDGE_R), 0)
    ohT_f = (niota == jnp.broadcast_to(f_row, (_NODE_R, _EDGE_R))).astype(jnp.float32)
    ohT_t = (niota == jnp.broadcast_to(t_row, (_NODE_R, _EDGE_R))).astype(jnp.float32)

    q_iota = jax.lax.broadcasted_iota(jnp.int32, (1, 64, 128), 1)
    c_iota = jax.lax.broadcasted_iota(jnp.int32, (1, 64, 128), 2) % 64
    valid3 = (q_iota < _EPG) & (c_iota < _EPG)
    lane_pad = jax.lax.broadcasted_iota(jnp.int32, (1, 1, 128), 2) % 64 < _EPG
    _H = _P // 2

    h = _mm(nf[...], Wen[...]) + ben[...]
    e = _mm(ef[...], Wee[...]) + bee[...]

    Ws1v = Ws1[...]
    bs1v = bs1[...]
    Ws2v = Ws2[...]
    bs2v = bs2[...]

    sub_pad4 = jax.lax.broadcasted_iota(jnp.int32, (1, 64, 1, 1), 1) < _EPG

    def _lse_half(x):
        x4 = x.reshape(_H, 64, 2, 64)
        m = jnp.max(x4, axis=3, keepdims=True)
        return m + jnp.log(jnp.sum(jnp.exp(x4 - m), axis=3, keepdims=True))

    def transport(e_all):
        sf = _mm(jax.nn.relu(_mm(e_all, Ws1v) + bs1v), Ws2v) + bs2v
        qs, cs, scores = [], [], []
        for p in range(_P):
            qf = sf[p * _EPP:p * _EPP + 64]
            cf = sf[p * _EPP + 64:(p + 1) * _EPP]
            qs.append(e_all[p * _EPP:p * _EPP + 64])
            cs.append(e_all[p * _EPP + 64:(p + 1) * _EPP])
            scores.append(_mm_bt(qf, cf))
        la = jnp.stack([jnp.concatenate([scores[2 * k], scores[2 * k + 1]],
                                        axis=1) for k in range(_H)])
        la = jnp.where(valid3, la, _NEG)
        v_row = None
        u_bc = None
        for it in range(_SINK_ITERS):
            u4 = -_lse_half(la if it == 0 else la + v_row)
            u4 = jnp.where(sub_pad4, u4, 0.0)
            u_bc = jnp.broadcast_to(u4, (_H, 64, 2, 64)).reshape(_H, 64, 128)
            v_row = jnp.where(lane_pad, -_lse(la + u_bc, 1), 0.0)
        T3 = jnp.exp(la + u_bc + v_row)
        qis, cis = [], []
        for k in range(_H):
            Tl = T3[k][:, :64]
            Tr = T3[k][:, 64:]
            qis.append(_mm(Tl, cs[2 * k]))
            qis.append(_mm(Tr, cs[2 * k + 1]))
            cis.append(_mm_at(Tl, qs[2 * k]))
            cis.append(_mm_at(Tr, qs[2 * k + 1]))
        return qs, qis, cis

    for _ in range(_PROP_STEPS):
        _, qis, cis = transport(e)
        pieces = []
        for p in range(_P):
            pieces.append(qis[p])
            pieces.append(cis[p])
        inter = jnp.concatenate(pieces, axis=0)
        h_f = _mm_at(ohT_f, h)
        h_t = _mm_at(ohT_t, h)
        msg = jax.nn.relu(_mm(h_f, Wm[0:64]) + _mm(h_t, Wm[64:128])
                          + _mm(e, Wm[128:160]) + bm[...])
        agg = _mm(ohT_t, msg)
        iagg = _mm(ohT_t, inter)
        h = jax.nn.relu(_mm(h, Wn[0:64]) + _mm(agg, Wn[64:96])
                        + _mm(agg - iagg, Wn[96:128]) + bn[...])
        h_f = _mm_at(ohT_f, h)
        h_t = _mm_at(ohT_t, h)
        e_new = jax.nn.relu(_mm(h_f, Wp[0:64]) + _mm(h_t, Wp[64:128])
                            + _mm(e, Wp[128:160]) + bp[...])
        e = _mm(jax.nn.relu(_mm(e_new, Wi1[0:32]) + _mm(inter, Wi1[32:64])
                            + bi1[...]), Wi2[...]) + bi2[...]

    qs, qis, _ = transport(e)
    scores = []
    for p in range(_P):
        scores.append(jnp.sum(qs[p] * qis[p]).reshape(1, 1))
    out[0] = jnp.concatenate(scores, axis=1)


@jax.jit
def _run(nf_pad, ef_pad, fl, tl,
         W_enc_n, b_enc_n, W_enc_e, b_enc_e, W_msg, b_msg, W_node, b_node,
         W_prop, b_prop, W_i1, b_i1, W_i2, b_i2, W_s1, b_s1, W_s2, b_s2):
    def wmap(_):
        return (0, 0)

    def w3map(i):
        return (i, 0, 0)

    def bmap(i):
        return (i, 0)

    grid = (_NBLK,)
    out = pl.pallas_call(
        _block_kernel,
        grid=grid,
        in_specs=[
            pl.BlockSpec((_NODE_R, 32), bmap),
            pl.BlockSpec((_EDGE_R, 16), bmap),
            pl.BlockSpec((1, 1, _EDGE_R), w3map),
            pl.BlockSpec((1, 1, _EDGE_R), w3map),
            pl.BlockSpec((32, 64), wmap),
            pl.BlockSpec((1, 64), wmap),
            pl.BlockSpec((16, 32), wmap),
            pl.BlockSpec((1, 32), wmap),
            pl.BlockSpec((160, 32), wmap),
            pl.BlockSpec((1, 32), wmap),
            pl.BlockSpec((128, 64), wmap),
            pl.BlockSpec((1, 64), wmap),
            pl.BlockSpec((160, 32), wmap),
            pl.BlockSpec((1, 32), wmap),
            pl.BlockSpec((64, 64), wmap),
            pl.BlockSpec((1, 64), wmap),
            pl.BlockSpec((64, 32), wmap),
            pl.BlockSpec((1, 32), wmap),
            pl.BlockSpec((32, 16), wmap),
            pl.BlockSpec((1, 16), wmap),
            pl.BlockSpec((16, 16), wmap),
            pl.BlockSpec((1, 16), wmap),
        ],
        out_specs=pl.BlockSpec((1, 1, _P), w3map),
        out_shape=jax.ShapeDtypeStruct((_NBLK, 1, _P), jnp.float32),
        compiler_params=pltpu.CompilerParams(
            dimension_semantics=("arbitrary",),
        ),
    )(nf_pad, ef_pad, fl, tl,
      W_enc_n, b_enc_n, W_enc_e, b_enc_e, W_msg, b_msg, W_node, b_node,
      W_prop, b_prop, W_i1, b_i1, W_i2, b_i2, W_s1, b_s1, W_s2, b_s2)
    return out.reshape(_B)


def kernel(node_features, edge_features, W_enc_n, b_enc_n, W_enc_e, b_enc_e,
           W_msg, b_msg, W_node, b_node, W_prop, b_prop,
           W_i1, b_i1, W_i2, b_i2, W_s1, b_s1, W_s2, b_s2,
           from_idx, to_idx, padded_edge_indices):
    garange = jnp.arange(_G, dtype=jnp.int32)

    def local_idx(idx):
        loc = idx.reshape(_G, _EPG) - (garange * _NPG)[:, None]
        loc = jnp.pad(loc, ((0, 0), (0, 4)), constant_values=_NPG)
        loc = loc + ((garange % 2) * 32)[:, None]
        loc = loc.reshape(_NBLK, _P, _EPP)
        loc = loc + (jnp.arange(_P, dtype=jnp.int32) * _NPP)[None, :, None]
        return loc.reshape(_NBLK, 1, _EDGE_R)

    fl = local_idx(from_idx)
    tl = local_idx(to_idx)

    nf_pad = jnp.pad(node_features.reshape(_G, _NPG, 32),
                     ((0, 0), (0, 2), (0, 0))).reshape(_NBLK * _NODE_R, 32)
    ef_pad = jnp.pad(edge_features.reshape(_G, _EPG, 16),
                     ((0, 0), (0, 4), (0, 0))).reshape(_NBLK * _EDGE_R, 16)

    s = (1.0 / _TEMP) ** 0.5
    return _run(nf_pad, ef_pad, fl, tl,
                W_enc_n, b_enc_n.reshape(1, -1), W_enc_e, b_enc_e.reshape(1, -1),
                W_msg, b_msg.reshape(1, -1), W_node, b_node.reshape(1, -1),
                W_prop, b_prop.reshape(1, -1), W_i1, b_i1.reshape(1, -1),
                W_i2, b_i2.reshape(1, -1), W_s1, b_s1.reshape(1, -1),
                W_s2 * s, b_s2.reshape(1, -1) * s)


# speedup vs baseline: 2.3613x; 2.3613x over previous
import functools

import jax
import jax.numpy as jnp
from jax.experimental import pallas as pl
from jax.experimental.pallas import tpu as pltpu

_B = 512
_NPG = 30
_EPG = 60
_G = 2 * _B
_N = _G * _NPG
_E = _G * _EPG
_PROP_STEPS = 3
_SINK_ITERS = 10
_TEMP = 0.1

_P = 8
_NBLK = _B // _P
_NPP = 64
_EPP = 128
_NODE_R = _P * _NPP
_EDGE_R = _P * _EPP
_NEG = -1e30


def _mm(a, b):
    return jax.lax.dot_general(a, b, (((1,), (0,)), ((), ())),
                               preferred_element_type=jnp.float32)


def _mm_bt(a, b):
    return jax.lax.dot_general(a, b, (((1,), (1,)), ((), ())),
                               preferred_element_type=jnp.float32)


def _mm_at(a, b):
    return jax.lax.dot_general(a, b, (((0,), (0,)), ((), ())),
                               preferred_element_type=jnp.float32)


def _lse(x, axis):
    m = jnp.max(x, axis=axis, keepdims=True)
    return m + jnp.log(jnp.sum(jnp.exp(x - m), axis=axis, keepdims=True))


def _block_kernel(nf, ef, fl, tl,
                  Wen, ben, Wee, bee, Wm, bm, Wn, bn, Wp, bp,
                  Wi1, bi1, Wi2, bi2, Ws1, bs1, Ws2, bs2, out):
    f_row = fl[0]
    t_row = tl[0]
    niota = jax.lax.broadcasted_iota(jnp.int32, (_NODE_R, _EDGE_R), 0)
    ohT_f = (niota == jnp.broadcast_to(f_row, (_NODE_R, _EDGE_R))).astype(jnp.float32)
    ohT_t = (niota == jnp.broadcast_to(t_row, (_NODE_R, _EDGE_R))).astype(jnp.float32)

    q_iota = jax.lax.broadcasted_iota(jnp.int32, (1, 64, 64), 1)
    c_iota = jax.lax.broadcasted_iota(jnp.int32, (1, 64, 64), 2)
    valid3 = (q_iota < _EPG) & (c_iota < _EPG)

    h = _mm(nf[...], Wen[...]) + ben[...]
    e = _mm(ef[...], Wee[...]) + bee[...]

    Ws1v = Ws1[...]
    bs1v = bs1[...]
    Ws2v = Ws2[...]
    bs2v = bs2[...]

    pad_vec = jax.lax.broadcasted_iota(jnp.int32, (1, 64, 1), 1) < _EPG

    def transport(e_all):
        sf = _mm(jax.nn.relu(_mm(e_all, Ws1v) + bs1v), Ws2v) + bs2v
        qs, cs, scores, scoresT = [], [], [], []
        for p in range(_P):
            qf = sf[p * _EPP:p * _EPP + 64]
            cf = sf[p * _EPP + 64:(p + 1) * _EPP]
            qs.append(e_all[p * _EPP:p * _EPP + 64])
            cs.append(e_all[p * _EPP + 64:(p + 1) * _EPP])
            scores.append(_mm_bt(qf, cf))
            scoresT.append(_mm_bt(cf, qf))
        la = jnp.where(valid3, jnp.stack(scores, axis=0) * (1.0 / _TEMP), _NEG)
        laT = jnp.where(valid3, jnp.stack(scoresT, axis=0) * (1.0 / _TEMP), _NEG)
        v_col = None
        for it in range(_SINK_ITERS):
            u_row = -_lse(laT if it == 0 else laT + v_col, 1)
            u_col = jnp.where(pad_vec, u_row.reshape(_P, 64, 1), 0.0)
            v_row = -_lse(la + u_col, 1)
            v_col = jnp.where(pad_vec, v_row.reshape(_P, 64, 1), 0.0)
        T3 = jnp.exp(la + u_col + v_col.reshape(_P, 1, 64))
        qis, cis = [], []
        for p in range(_P):
            Tp = T3[p]
            qis.append(_mm(Tp, cs[p]))
            cis.append(_mm_at(Tp, qs[p]))
        return qs, qis, cis

    for _ in range(_PROP_STEPS):
        _, qis, cis = transport(e)
        pieces = []
        for p in range(_P):
            pieces.append(qis[p])
            pieces.append(cis[p])
        inter = jnp.concatenate(pieces, axis=0)
        h_f = _mm_at(ohT_f, h)
        h_t = _mm_at(ohT_t, h)
        msg = jax.nn.relu(_mm(h_f, Wm[0:64]) + _mm(h_t, Wm[64:128])
                          + _mm(e, Wm[128:160]) + bm[...])
        agg = _mm(ohT_t, msg)
        iagg = _mm(ohT_t, inter)
        h = jax.nn.relu(_mm(h, Wn[0:64]) + _mm(agg, Wn[64:96])
                        + _mm(agg - iagg, Wn[96:128]) + bn[...])
        h_f = _mm_at(ohT_f, h)
        h_t = _mm_at(ohT_t, h)
        e_new = jax.nn.relu(_mm(h_f, Wp[0:64]) + _mm(h_t, Wp[64:128])
                            + _mm(e, Wp[128:160]) + bp[...])
        e = _mm(jax.nn.relu(_mm(e_new, Wi1[0:32]) + _mm(inter, Wi1[32:64])
                            + bi1[...]), Wi2[...]) + bi2[...]

    qs, qis, _ = transport(e)
    scores = []
    for p in range(_P):
        scores.append(jnp.sum(qs[p] * qis[p]).reshape(1, 1))
    out[0] = jnp.concatenate(scores, axis=1)


@jax.jit
def _run(nf_pad, ef_pad, fl, tl,
         W_enc_n, b_enc_n, W_enc_e, b_enc_e, W_msg, b_msg, W_node, b_node,
         W_prop, b_prop, W_i1, b_i1, W_i2, b_i2, W_s1, b_s1, W_s2, b_s2):
    def wmap(_):
        return (0, 0)

    def w3map(i):
        return (i, 0, 0)

    def bmap(i):
        return (i, 0)

    grid = (_NBLK,)
    out = pl.pallas_call(
        _block_kernel,
        grid=grid,
        in_specs=[
            pl.BlockSpec((_NODE_R, 32), bmap),
            pl.BlockSpec((_EDGE_R, 16), bmap),
            pl.BlockSpec((1, 1, _EDGE_R), w3map),
            pl.BlockSpec((1, 1, _EDGE_R), w3map),
            pl.BlockSpec((32, 64), wmap),
            pl.BlockSpec((1, 64), wmap),
            pl.BlockSpec((16, 32), wmap),
            pl.BlockSpec((1, 32), wmap),
            pl.BlockSpec((160, 32), wmap),
            pl.BlockSpec((1, 32), wmap),
            pl.BlockSpec((128, 64), wmap),
            pl.BlockSpec((1, 64), wmap),
            pl.BlockSpec((160, 32), wmap),
            pl.BlockSpec((1, 32), wmap),
            pl.BlockSpec((64, 64), wmap),
            pl.BlockSpec((1, 64), wmap),
            pl.BlockSpec((64, 32), wmap),
            pl.BlockSpec((1, 32), wmap),
            pl.BlockSpec((32, 16), wmap),
            pl.BlockSpec((1, 16), wmap),
            pl.BlockSpec((16, 16), wmap),
            pl.BlockSpec((1, 16), wmap),
        ],
        out_specs=pl.BlockSpec((1, 1, _P), w3map),
        out_shape=jax.ShapeDtypeStruct((_NBLK, 1, _P), jnp.float32),
        compiler_params=pltpu.CompilerParams(
            dimension_semantics=("arbitrary",),
        ),
    )(nf_pad, ef_pad, fl, tl,
      W_enc_n, b_enc_n, W_enc_e, b_enc_e, W_msg, b_msg, W_node, b_node,
      W_prop, b_prop, W_i1, b_i1, W_i2, b_i2, W_s1, b_s1, W_s2, b_s2)
    return out.reshape(_B)


def kernel(node_features, edge_features, W_enc_n, b_enc_n, W_enc_e, b_enc_e,
           W_msg, b_msg, W_node, b_node, W_prop, b_prop,
           W_i1, b_i1, W_i2, b_i2, W_s1, b_s1, W_s2, b_s2,
           from_idx, to_idx, padded_edge_indices):
    garange = jnp.arange(_G, dtype=jnp.int32)

    def local_idx(idx):
        loc = idx.reshape(_G, _EPG) - (garange * _NPG)[:, None]
        loc = jnp.pad(loc, ((0, 0), (0, 4)), constant_values=_NPG)
        loc = loc + ((garange % 2) * 32)[:, None]
        loc = loc.reshape(_NBLK, _P, _EPP)
        loc = loc + (jnp.arange(_P, dtype=jnp.int32) * _NPP)[None, :, None]
        return loc.reshape(_NBLK, 1, _EDGE_R)

    fl = local_idx(from_idx)
    tl = local_idx(to_idx)

    nf_pad = jnp.pad(node_features.reshape(_G, _NPG, 32),
                     ((0, 0), (0, 2), (0, 0))).reshape(_NBLK * _NODE_R, 32)
    ef_pad = jnp.pad(edge_features.reshape(_G, _EPG, 16),
                     ((0, 0), (0, 4), (0, 0))).reshape(_NBLK * _EDGE_R, 16)

    return _run(nf_pad, ef_pad, fl, tl,
                W_enc_n, b_enc_n.reshape(1, -1), W_enc_e, b_enc_e.reshape(1, -1),
                W_msg, b_msg.reshape(1, -1), W_node, b_node.reshape(1, -1),
                W_prop, b_prop.reshape(1, -1), W_i1, b_i1.reshape(1, -1),
                W_i2, b_i2.reshape(1, -1), W_s1, b_s1.reshape(1, -1),
                W_s2, b_s2.reshape(1, -1))
